# Initial kernel scaffold; baseline (speedup 1.0000x reference)
import jax
import jax.numpy as jnp
from jax.experimental import pallas as pl


def kernel(cls_pred, loc_pred):
    raise NotImplementedError("write your pallas kernel here")


# speedup vs baseline: 4.2140x; 4.2140x over previous
import functools

import jax
import jax.numpy as jnp
from jax import lax
from jax.experimental import pallas as pl
from jax.experimental.pallas import tpu as pltpu
from jax.experimental.pallas import tpu_sc as plsc

L = 16
N = 1310720
NW = 16
CHUNK = N // NW
ROWS = CHUNK // L
G1 = ROWS // L
G2 = G1 // L
K = 100
KPAD = 112
MERGE = NW * KPAD
MG1 = MERGE // L // L
NUM_CLASSES = 80
BIG = 2**30
NEG = float("-inf")


def _vmax_fold(ref, base, n):
    acc = ref[pl.ds(base, L)]
    for j in range(1, n):
        acc = jnp.maximum(acc, ref[pl.ds(base + j * L, L)])
    return acc


def _first_match(ref, base_vec_id, n, s):
    acc = jnp.full((L,), BIG, jnp.int32)
    for j in range(n):
        v = ref[pl.ds((base_vec_id + j) * L, L)]
        acc = jnp.minimum(acc, jnp.where(v == s, jnp.full((L,), base_vec_id + j, jnp.int32), BIG))
    return jnp.min(acc)


def _lane0_scatter(ref, pos, val):
    iota = lax.iota(jnp.int32, L)
    plsc.store_scatter(ref, [jnp.full((L,), pos, jnp.int32)],
                       jnp.full((L,), val), mask=iota == 0)


def _extract_topk(data, t1, t2, n2, fan, vals, idxs, k, idx_of_pos):
    iota = lax.iota(jnp.int32, L)

    def body(i, _):
        m = _vmax_fold(t2, 0, n2)
        s = jnp.max(m)
        h = _first_match(t2, 0, n2, s)
        g = _first_match(t1, h * fan, fan, s)
        r = _first_match(data, g * L, L, s)
        d = data[pl.ds(r * L, L)]
        lane = jnp.min(jnp.where(d == s, iota, BIG))
        pos = r * L + lane
        _lane0_scatter(vals, i, s)
        _lane0_scatter(idxs, i, idx_of_pos(pos))
        data[pl.ds(r * L, L)] = jnp.where(iota == lane, NEG, d)
        t1[pl.ds(g * L, L)] = _vmax_fold(data, g * L * L, L)
        t2[pl.ds(h * L, L)] = _vmax_fold(t1, h * fan * L, fan)
        return 0

    lax.fori_loop(0, k, body, 0)


def _decode_kernel(cls_hbm, loc_hbm, out_hbm,
                   data_v, t1_v, t2_v, vals_v, idx_v,
                   shv, shi, mv_v, mi_v, mt1_v, mt2_v,
                   fv_v, fi_v, spat_v, rows_v, det_v, sem):
    cid = lax.axis_index("c")
    sid = lax.axis_index("s")
    iota = lax.iota(jnp.int32, L)

    @pl.when(cid == 0)
    def _core0():
        base = sid * CHUNK
        pltpu.sync_copy(cls_hbm.at[pl.ds(base, CHUNK)], data_v)

        def f1(g, _):
            t1_v[pl.ds(g * L, L)] = _vmax_fold(data_v, g * L * L, L)
            return 0
        lax.fori_loop(0, G1, f1, 0)

        def f2(h, _):
            t2_v[pl.ds(h * L, L)] = _vmax_fold(t1_v, h * L * L, L)
            return 0
        lax.fori_loop(0, G2, f2, 0)

        _extract_topk(data_v, t1_v, t2_v, G2, L, vals_v, idx_v, K,
                      lambda p: base + p)

        tail = vals_v[pl.ds(K - 4, L)]
        vals_v[pl.ds(K - 4, L)] = jnp.where(iota < 4, tail, NEG)
        taili = idx_v[pl.ds(K - 4, L)]
        idx_v[pl.ds(K - 4, L)] = jnp.where(iota < 4, taili, BIG)

        pltpu.sync_copy(vals_v, shv.at[pl.ds(sid * KPAD, KPAD)])
        pltpu.sync_copy(idx_v, shi.at[pl.ds(sid * KPAD, KPAD)])
        plsc.subcore_barrier()

        @pl.when(sid == 0)
        def _merge():
            pltpu.sync_copy(shv, mv_v)
            pltpu.sync_copy(shi, mi_v)

            for g in range(MG1):
                mt1_v[pl.ds(g * L, L)] = _vmax_fold(mv_v, g * L * L, L)
            mt2 = _vmax_fold(mt1_v, 0, MG1)
            mt2_v[pl.ds(0, L)] = mt2

            def idx_of(p):
                dv = mi_v[pl.ds((p // L) * L, L)]
                return jnp.min(jnp.where(iota == (p % L), dv, BIG))

            _extract_topk(mv_v, mt1_v, mt2_v, 1, MG1, fv_v, fi_v, K, idx_of)

            zeros = jnp.zeros((L,), jnp.int32)
            for j in range(4):
                for c in range(KPAD // L):
                    p = c * L + iota
                    gidx = fi_v[pl.ds(c * L, L)]
                    eidx = (gidx // NUM_CLASSES) * 4 + j
                    spat_v[pl.ds(c * L, L)] = jnp.where(p < K, eidx, 0)
                pltpu.async_copy(loc_hbm.at[spat_v], rows_v.at[j], sem).wait()

            for c in range(KPAD // L):
                p = c * L + iota
                valid = p < K
                gidx = fi_v[pl.ds(c * L, L)]
                for j in range(4):
                    colv = rows_v[j, pl.ds(c * L, L)]
                    plsc.store_scatter(det_v, [p, zeros + j], colv * 4.0, mask=valid)
                sc = fv_v[pl.ds(c * L, L)]
                plsc.store_scatter(det_v, [p, zeros + 4], sc, mask=valid)
                clsf = (gidx % NUM_CLASSES).astype(jnp.float32)
                plsc.store_scatter(det_v, [p, zeros + 5], clsf, mask=valid)

            pltpu.sync_copy(det_v, out_hbm)


@jax.jit
def kernel(cls_pred, loc_pred):
    cls_flat = jnp.reshape(cls_pred, (N,))
    loc_flat = jnp.reshape(loc_pred, (N // NUM_CLASSES * 4,))

    run = pl.kernel(
        _decode_kernel,
        out_type=jax.ShapeDtypeStruct((K, 6), jnp.float32),
        mesh=plsc.VectorSubcoreMesh(core_axis_name="c", subcore_axis_name="s"),
        compiler_params=pltpu.CompilerParams(needs_layout_passes=False),
        scratch_types=[
            pltpu.VMEM((CHUNK,), jnp.float32),
            pltpu.VMEM((ROWS,), jnp.float32),
            pltpu.VMEM((G1,), jnp.float32),
            pltpu.VMEM((KPAD,), jnp.float32),
            pltpu.VMEM((KPAD,), jnp.int32),
            pltpu.VMEM_SHARED((MERGE,), jnp.float32),
            pltpu.VMEM_SHARED((MERGE,), jnp.int32),
            pltpu.VMEM((MERGE,), jnp.float32),
            pltpu.VMEM((MERGE,), jnp.int32),
            pltpu.VMEM((MG1 * L,), jnp.float32),
            pltpu.VMEM((L,), jnp.float32),
            pltpu.VMEM((KPAD,), jnp.float32),
            pltpu.VMEM((KPAD,), jnp.int32),
            pltpu.VMEM((KPAD,), jnp.int32),
            pltpu.VMEM((4, KPAD), jnp.float32),
            pltpu.VMEM((K, 6), jnp.float32),
            pltpu.SemaphoreType.DMA,
        ],
    )
    det = run(cls_flat, loc_flat)
    return jnp.reshape(det, (1, K, 6))


# speedup vs baseline: 4.8558x; 1.1523x over previous
import functools

import jax
import jax.numpy as jnp
from jax import lax
from jax.experimental import pallas as pl
from jax.experimental.pallas import tpu as pltpu
from jax.experimental.pallas import tpu_sc as plsc

L = 16
N = 1310720
NW = 16
CHUNK = N // NW
ROWS = CHUNK // L
G1 = ROWS // L
G2 = G1 // L
K = 100
KPAD = 112
ROUND = 16
MERGE = NW * KPAD
MG1 = MERGE // L // L
NUM_CLASSES = 80
BIG = 2**30
NEG = float("-inf")


def _vmax_fold(ref, base, n):
    acc = ref[pl.ds(base, L)]
    for j in range(1, n):
        acc = jnp.maximum(acc, ref[pl.ds(base + j * L, L)])
    return acc


def _vadd_fold(ref, base, n):
    acc = ref[pl.ds(base, L)]
    for j in range(1, n):
        acc = acc + ref[pl.ds(base + j * L, L)]
    return acc


def _first_match(ref, base_vec_id, n, s):
    acc = jnp.full((L,), BIG, jnp.int32)
    for j in range(n):
        v = ref[pl.ds((base_vec_id + j) * L, L)]
        acc = jnp.minimum(acc, jnp.where(v == s, jnp.full((L,), base_vec_id + j, jnp.int32), BIG))
    return jnp.min(acc)


def _lane0_scatter(ref, pos, val):
    iota = lax.iota(jnp.int32, L)
    plsc.store_scatter(ref, [jnp.full((L,), pos, jnp.int32)],
                       jnp.full((L,), val), mask=iota == 0)


def _extract_topk(data, t1, t2, n2, fan, vals, idxs, k, idx_of_pos, start=0):
    iota = lax.iota(jnp.int32, L)

    def body(i, _):
        m = _vmax_fold(t2, 0, n2)
        s = jnp.max(m)
        h = _first_match(t2, 0, n2, s)
        g = _first_match(t1, h * fan, fan, s)
        r = _first_match(data, g * L, L, s)
        d = data[pl.ds(r * L, L)]
        lane = jnp.min(jnp.where(d == s, iota, BIG))
        pos = r * L + lane
        _lane0_scatter(vals, start + i, s)
        _lane0_scatter(idxs, start + i, idx_of_pos(pos))
        data[pl.ds(r * L, L)] = jnp.where(iota == lane, NEG, d)
        t1[pl.ds(g * L, L)] = _vmax_fold(data, g * L * L, L)
        t2[pl.ds(h * L, L)] = _vmax_fold(t1, h * fan * L, fan)
        return 0

    lax.fori_loop(0, k, body, 0)


def _decode_kernel(cls_hbm, loc_hbm, out_hbm,
                   data_v, t1_v, t2_v, vals_v, idx_v,
                   shv, shi, shr, shc, rb_v, rall_v, cb_v, call_v,
                   mv_v, mi_v, mt1_v, mt2_v,
                   fv_v, fi_v, spat_v, rows_v, det_v, sem):
    cid = lax.axis_index("c")
    sid = lax.axis_index("s")
    iota = lax.iota(jnp.int32, L)

    @pl.when(cid == 0)
    def _core0():
        base = sid * CHUNK
        pltpu.sync_copy(cls_hbm.at[pl.ds(base, CHUNK)], data_v)

        def f1(g, _):
            t1_v[pl.ds(g * L, L)] = _vmax_fold(data_v, g * L * L, L)
            return 0
        lax.fori_loop(0, G1, f1, 0)

        def f2(h, _):
            t2_v[pl.ds(h * L, L)] = _vmax_fold(t1_v, h * L * L, L)
            return 0
        lax.fori_loop(0, G2, f2, 0)

        for c in range(KPAD // L):
            vals_v[pl.ds(c * L, L)] = jnp.full((L,), NEG, jnp.float32)
            idx_v[pl.ds(c * L, L)] = jnp.full((L,), BIG, jnp.int32)

        def round_body(state):
            n_ext, _ = state
            _extract_topk(data_v, t1_v, t2_v, G2, L, vals_v, idx_v, ROUND,
                          lambda p: base + p, start=n_ext)
            pltpu.sync_copy(vals_v, shv.at[pl.ds(sid * KPAD, KPAD)])
            pltpu.sync_copy(idx_v, shi.at[pl.ds(sid * KPAD, KPAD)])
            rem = jnp.max(_vmax_fold(t2_v, 0, G2))
            rb_v[pl.ds(0, L)] = jnp.full((L,), rem)
            pltpu.sync_copy(rb_v, shr.at[pl.ds(sid * L, L)])
            plsc.subcore_barrier()
            pltpu.sync_copy(shr, rall_v)
            gr = jnp.max(_vmax_fold(rall_v, 0, NW))
            acc = jnp.zeros((L,), jnp.int32)
            for c in range(KPAD // L):
                acc = acc + (vals_v[pl.ds(c * L, L)] > gr).astype(jnp.int32)
            cb_v[pl.ds(0, L)] = jnp.full((L,), jnp.sum(acc))
            pltpu.sync_copy(cb_v, shc.at[pl.ds(sid * L, L)])
            plsc.subcore_barrier()
            pltpu.sync_copy(shc, call_v)
            tot = jnp.max(_vadd_fold(call_v, 0, NW))
            n_new = n_ext + ROUND
            return n_new, jnp.logical_and(tot < K, n_new + ROUND <= KPAD)

        lax.while_loop(lambda st: st[1], round_body, (0, True))

        @pl.when(sid == 0)
        def _merge():
            pltpu.sync_copy(shv, mv_v)
            pltpu.sync_copy(shi, mi_v)

            for g in range(MG1):
                mt1_v[pl.ds(g * L, L)] = _vmax_fold(mv_v, g * L * L, L)
            mt2 = _vmax_fold(mt1_v, 0, MG1)
            mt2_v[pl.ds(0, L)] = mt2

            def idx_of(p):
                dv = mi_v[pl.ds((p // L) * L, L)]
                return jnp.min(jnp.where(iota == (p % L), dv, BIG))

            _extract_topk(mv_v, mt1_v, mt2_v, 1, MG1, fv_v, fi_v, K, idx_of)

            zeros = jnp.zeros((L,), jnp.int32)
            for j in range(4):
                for c in range(KPAD // L):
                    p = c * L + iota
                    gidx = fi_v[pl.ds(c * L, L)]
                    eidx = (gidx // NUM_CLASSES) * 4 + j
                    spat_v[pl.ds(c * L, L)] = jnp.where(p < K, eidx, 0)
                pltpu.async_copy(loc_hbm.at[spat_v], rows_v.at[j], sem).wait()

            for c in range(KPAD // L):
                p = c * L + iota
                valid = p < K
                gidx = fi_v[pl.ds(c * L, L)]
                for j in range(4):
                    colv = rows_v[j, pl.ds(c * L, L)]
                    plsc.store_scatter(det_v, [p, zeros + j], colv * 4.0, mask=valid)
                sc = fv_v[pl.ds(c * L, L)]
                plsc.store_scatter(det_v, [p, zeros + 4], sc, mask=valid)
                clsf = (gidx % NUM_CLASSES).astype(jnp.float32)
                plsc.store_scatter(det_v, [p, zeros + 5], clsf, mask=valid)

            pltpu.sync_copy(det_v, out_hbm)


@jax.jit
def kernel(cls_pred, loc_pred):
    cls_flat = jnp.reshape(cls_pred, (N,))
    loc_flat = jnp.reshape(loc_pred, (N // NUM_CLASSES * 4,))

    run = pl.kernel(
        _decode_kernel,
        out_type=jax.ShapeDtypeStruct((K, 6), jnp.float32),
        mesh=plsc.VectorSubcoreMesh(core_axis_name="c", subcore_axis_name="s"),
        compiler_params=pltpu.CompilerParams(needs_layout_passes=False),
        scratch_types=[
            pltpu.VMEM((CHUNK,), jnp.float32),
            pltpu.VMEM((ROWS,), jnp.float32),
            pltpu.VMEM((G1,), jnp.float32),
            pltpu.VMEM((KPAD,), jnp.float32),
            pltpu.VMEM((KPAD,), jnp.int32),
            pltpu.VMEM_SHARED((MERGE,), jnp.float32),
            pltpu.VMEM_SHARED((MERGE,), jnp.int32),
            pltpu.VMEM_SHARED((NW * L,), jnp.float32),
            pltpu.VMEM_SHARED((NW * L,), jnp.int32),
            pltpu.VMEM((L,), jnp.float32),
            pltpu.VMEM((NW * L,), jnp.float32),
            pltpu.VMEM((L,), jnp.int32),
            pltpu.VMEM((NW * L,), jnp.int32),
            pltpu.VMEM((MERGE,), jnp.float32),
            pltpu.VMEM((MERGE,), jnp.int32),
            pltpu.VMEM((MG1 * L,), jnp.float32),
            pltpu.VMEM((L,), jnp.float32),
            pltpu.VMEM((KPAD,), jnp.float32),
            pltpu.VMEM((KPAD,), jnp.int32),
            pltpu.VMEM((KPAD,), jnp.int32),
            pltpu.VMEM((4, KPAD), jnp.float32),
            pltpu.VMEM((K, 6), jnp.float32),
            pltpu.SemaphoreType.DMA,
        ],
    )
    det = run(cls_flat, loc_flat)
    return jnp.reshape(det, (1, K, 6))


# speedup vs baseline: 4.9317x; 1.0156x over previous
import functools

import jax
import jax.numpy as jnp
from jax import lax
from jax.experimental import pallas as pl
from jax.experimental.pallas import tpu as pltpu
from jax.experimental.pallas import tpu_sc as plsc

L = 16
N = 1310720
NW = 16
CHUNK = N // NW
ROWS = CHUNK // L
G1 = ROWS // L
G2 = G1 // L
K = 100
KPAD = 112
ROUND = 16
MERGE = NW * KPAD
MG1 = MERGE // L // L
NUM_CLASSES = 80
BIG = 2**30
NEG = float("-inf")


def _vmax_fold(ref, base, n):
    acc = ref[pl.ds(base, L)]
    for j in range(1, n):
        acc = jnp.maximum(acc, ref[pl.ds(base + j * L, L)])
    return acc


def _vadd_fold(ref, base, n):
    acc = ref[pl.ds(base, L)]
    for j in range(1, n):
        acc = acc + ref[pl.ds(base + j * L, L)]
    return acc


def _first_match(ref, base_vec_id, n, s):
    acc = jnp.full((L,), BIG, jnp.int32)
    for j in range(n):
        v = ref[pl.ds((base_vec_id + j) * L, L)]
        acc = jnp.minimum(acc, jnp.where(v == s, jnp.full((L,), base_vec_id + j, jnp.int32), BIG))
    return jnp.min(acc)


def _lane0_scatter(ref, pos, val):
    iota = lax.iota(jnp.int32, L)
    plsc.store_scatter(ref, [jnp.full((L,), pos, jnp.int32)],
                       jnp.full((L,), val), mask=iota == 0)


def _extract_topk(data, t1, t2, n2, fan, vals, idxs, k, idx_of_pos, start=0):
    iota = lax.iota(jnp.int32, L)

    def body(i, _):
        m = _vmax_fold(t2, 0, n2)
        s = jnp.max(m)
        h = _first_match(t2, 0, n2, s)
        g = _first_match(t1, h * fan, fan, s)
        r = _first_match(data, g * L, L, s)
        d = data[pl.ds(r * L, L)]
        lane = jnp.min(jnp.where(d == s, iota, BIG))
        pos = r * L + lane
        _lane0_scatter(vals, start + i, s)
        _lane0_scatter(idxs, start + i, idx_of_pos(pos))
        data[pl.ds(r * L, L)] = jnp.where(iota == lane, NEG, d)
        t1[pl.ds(g * L, L)] = _vmax_fold(data, g * L * L, L)
        t2[pl.ds(h * L, L)] = _vmax_fold(t1, h * fan * L, fan)
        return 0

    lax.fori_loop(0, k, body, 0)


def _decode_kernel(cls_hbm, loc_hbm, out_hbm,
                   data_v, t1_v, t2_v, vals_v, idx_v,
                   shv, shi, shr, shc, rb_v, rall_v, cb_v, call_v,
                   mv_v, mi_v, mt1_v, mt2_v,
                   fv_v, fi_v, spat_v, rows_v, det_v, sem):
    cid = lax.axis_index("c")
    sid = lax.axis_index("s")
    iota = lax.iota(jnp.int32, L)

    @pl.when(cid == 0)
    def _core0():
        base = sid * CHUNK
        pltpu.sync_copy(cls_hbm.at[pl.ds(base, CHUNK)], data_v)

        def f1(g, _):
            t1_v[pl.ds(g * L, L)] = _vmax_fold(data_v, g * L * L, L)
            return 0
        lax.fori_loop(0, G1, f1, 0)

        def f2(h, _):
            t2_v[pl.ds(h * L, L)] = _vmax_fold(t1_v, h * L * L, L)
            return 0
        lax.fori_loop(0, G2, f2, 0)

        for c in range(KPAD // L):
            vals_v[pl.ds(c * L, L)] = jnp.full((L,), NEG, jnp.float32)
            idx_v[pl.ds(c * L, L)] = jnp.full((L,), BIG, jnp.int32)

        def round_body(state):
            n_ext, _ = state
            _extract_topk(data_v, t1_v, t2_v, G2, L, vals_v, idx_v, ROUND,
                          lambda p: base + p, start=n_ext)
            pltpu.sync_copy(vals_v, shv.at[pl.ds(sid * KPAD, KPAD)])
            pltpu.sync_copy(idx_v, shi.at[pl.ds(sid * KPAD, KPAD)])
            rem = jnp.max(_vmax_fold(t2_v, 0, G2))
            rb_v[pl.ds(0, L)] = jnp.full((L,), rem)
            pltpu.sync_copy(rb_v, shr.at[pl.ds(sid * L, L)])
            plsc.subcore_barrier()
            pltpu.sync_copy(shr, rall_v)
            gr = jnp.max(_vmax_fold(rall_v, 0, NW))
            acc = jnp.zeros((L,), jnp.int32)
            for c in range(KPAD // L):
                acc = acc + (vals_v[pl.ds(c * L, L)] > gr).astype(jnp.int32)
            cb_v[pl.ds(0, L)] = jnp.full((L,), jnp.sum(acc))
            pltpu.sync_copy(cb_v, shc.at[pl.ds(sid * L, L)])
            plsc.subcore_barrier()
            pltpu.sync_copy(shc, call_v)
            tot = jnp.max(_vadd_fold(call_v, 0, NW))
            n_new = n_ext + ROUND
            return n_new, jnp.logical_and(tot < K, n_new + ROUND <= KPAD)

        lax.while_loop(lambda st: st[1], round_body, (0, True))

        @pl.when(sid == 0)
        def _merge():
            pltpu.sync_copy(shv, mv_v)
            pltpu.sync_copy(shi, mi_v)

            for g in range(MG1):
                mt1_v[pl.ds(g * L, L)] = _vmax_fold(mv_v, g * L * L, L)
            mt2 = _vmax_fold(mt1_v, 0, MG1)
            mt2_v[pl.ds(0, L)] = mt2

            def idx_of(p):
                dv = mi_v[pl.ds((p // L) * L, L)]
                return jnp.min(jnp.where(iota == (p % L), dv, BIG))

            _extract_topk(mv_v, mt1_v, mt2_v, 1, MG1, fv_v, fi_v, K, idx_of)

            zeros = jnp.zeros((L,), jnp.int32)
            for j in range(4):
                for c in range(KPAD // L):
                    p = c * L + iota
                    gidx = fi_v[pl.ds(c * L, L)]
                    eidx = (gidx // NUM_CLASSES) * 4 + j
                    spat_v[pl.ds(c * L, L)] = jnp.where(p < K, eidx, 0)
                pltpu.async_copy(loc_hbm.at[spat_v], rows_v.at[j], sem).wait()

            for c in range(KPAD // L):
                p = c * L + iota
                valid = p < K
                gidx = fi_v[pl.ds(c * L, L)]
                for j in range(4):
                    colv = rows_v[j, pl.ds(c * L, L)]
                    plsc.store_scatter(det_v, [p, zeros + j], colv * 4.0, mask=valid)
                sc = fv_v[pl.ds(c * L, L)]
                plsc.store_scatter(det_v, [p, zeros + 4], sc, mask=valid)
                clsf = (gidx % NUM_CLASSES).astype(jnp.float32)
                plsc.store_scatter(det_v, [p, zeros + 5], clsf, mask=valid)

            pltpu.sync_copy(det_v, out_hbm)


@jax.jit
def kernel(cls_pred, loc_pred):
    cls_flat = jnp.reshape(cls_pred, (N,))
    loc_flat = jnp.reshape(loc_pred, (N // NUM_CLASSES * 4,))

    run = pl.kernel(
        _decode_kernel,
        out_type=jax.ShapeDtypeStruct((K, 6), jnp.float32),
        mesh=plsc.VectorSubcoreMesh(core_axis_name="c", subcore_axis_name="s",
                                    num_cores=1),
        compiler_params=pltpu.CompilerParams(needs_layout_passes=False),
        scratch_types=[
            pltpu.VMEM((CHUNK,), jnp.float32),
            pltpu.VMEM((ROWS,), jnp.float32),
            pltpu.VMEM((G1,), jnp.float32),
            pltpu.VMEM((KPAD,), jnp.float32),
            pltpu.VMEM((KPAD,), jnp.int32),
            pltpu.VMEM_SHARED((MERGE,), jnp.float32),
            pltpu.VMEM_SHARED((MERGE,), jnp.int32),
            pltpu.VMEM_SHARED((NW * L,), jnp.float32),
            pltpu.VMEM_SHARED((NW * L,), jnp.int32),
            pltpu.VMEM((L,), jnp.float32),
            pltpu.VMEM((NW * L,), jnp.float32),
            pltpu.VMEM((L,), jnp.int32),
            pltpu.VMEM((NW * L,), jnp.int32),
            pltpu.VMEM((MERGE,), jnp.float32),
            pltpu.VMEM((MERGE,), jnp.int32),
            pltpu.VMEM((MG1 * L,), jnp.float32),
            pltpu.VMEM((L,), jnp.float32),
            pltpu.VMEM((KPAD,), jnp.float32),
            pltpu.VMEM((KPAD,), jnp.int32),
            pltpu.VMEM((KPAD,), jnp.int32),
            pltpu.VMEM((4, KPAD), jnp.float32),
            pltpu.VMEM((K, 6), jnp.float32),
            pltpu.SemaphoreType.DMA,
        ],
    )
    det = run(cls_flat, loc_flat)
    return jnp.reshape(det, (1, K, 6))


# speedup vs baseline: 5.1268x; 1.0396x over previous
import functools

import jax
import jax.numpy as jnp
from jax import lax
from jax.experimental import pallas as pl
from jax.experimental.pallas import tpu as pltpu
from jax.experimental.pallas import tpu_sc as plsc

L = 16
N = 1310720
NW = 16
CHUNK = N // NW
ROWS = CHUNK // L
G1 = ROWS // L
G2 = G1 // L
K = 100
KPAD = 112
ROUND = 16
MERGE = NW * KPAD
MG1 = MERGE // L // L
NUM_CLASSES = 80
BIG = 2**30
NEG = float("-inf")


def _vmax_fold(ref, base, n):
    acc = ref[pl.ds(base, L)]
    for j in range(1, n):
        acc = jnp.maximum(acc, ref[pl.ds(base + j * L, L)])
    return acc


def _vadd_fold(ref, base, n):
    acc = ref[pl.ds(base, L)]
    for j in range(1, n):
        acc = acc + ref[pl.ds(base + j * L, L)]
    return acc


def _first_match(ref, base_vec_id, n, s):
    acc = jnp.full((L,), BIG, jnp.int32)
    for j in range(n):
        v = ref[pl.ds((base_vec_id + j) * L, L)]
        acc = jnp.minimum(acc, jnp.where(v == s, jnp.full((L,), base_vec_id + j, jnp.int32), BIG))
    return jnp.min(acc)


def _lane0_scatter(ref, pos, val):
    iota = lax.iota(jnp.int32, L)
    plsc.store_scatter(ref, [jnp.full((L,), pos, jnp.int32)],
                       jnp.full((L,), val), mask=iota == 0)


def _extract_topk(data, t1, t2, n2, fan, vals, idxs, k, idx_of_pos, start=0):
    iota = lax.iota(jnp.int32, L)

    def body(i, _):
        m = _vmax_fold(t2, 0, n2)
        s = jnp.max(m)
        h = _first_match(t2, 0, n2, s)
        g = _first_match(t1, h * fan, fan, s)
        acc = jnp.full((L,), BIG, jnp.int32)
        for j in range(L):
            d = data[pl.ds((g * L + j) * L, L)]
            acc = jnp.minimum(acc, jnp.where(d == s, (g * L + j) * L + iota, BIG))
        pos = jnp.min(acc)
        r = pos // L
        lane = pos % L
        _lane0_scatter(vals, start + i, s)
        _lane0_scatter(idxs, start + i, idx_of_pos(pos))
        d = data[pl.ds(r * L, L)]
        data[pl.ds(r * L, L)] = jnp.where(iota == lane, NEG, d)
        t1[pl.ds(g * L, L)] = _vmax_fold(data, g * L * L, L)
        t2[pl.ds(h * L, L)] = _vmax_fold(t1, h * fan * L, fan)
        return 0

    lax.fori_loop(0, k, body, 0)


def _decode_kernel(cls_hbm, loc_hbm, out_hbm,
                   data_v, t1_v, t2_v, vals_v, idx_v,
                   shv, shi, shr, shc, rb_v, rall_v, cb_v, call_v,
                   mv_v, mi_v, mt1_v, mt2_v,
                   fv_v, fi_v, spat_v, rows_v, det_v, sem):
    cid = lax.axis_index("c")
    sid = lax.axis_index("s")
    iota = lax.iota(jnp.int32, L)

    @pl.when(cid == 0)
    def _core0():
        base = sid * CHUNK
        pltpu.sync_copy(cls_hbm.at[pl.ds(base, CHUNK)], data_v)

        def f1(g, _):
            t1_v[pl.ds(g * L, L)] = _vmax_fold(data_v, g * L * L, L)
            return 0
        lax.fori_loop(0, G1, f1, 0)

        def f2(h, _):
            t2_v[pl.ds(h * L, L)] = _vmax_fold(t1_v, h * L * L, L)
            return 0
        lax.fori_loop(0, G2, f2, 0)

        for c in range(KPAD // L):
            vals_v[pl.ds(c * L, L)] = jnp.full((L,), NEG, jnp.float32)
            idx_v[pl.ds(c * L, L)] = jnp.full((L,), BIG, jnp.int32)

        def round_body(state):
            n_ext, _ = state
            _extract_topk(data_v, t1_v, t2_v, G2, L, vals_v, idx_v, ROUND,
                          lambda p: base + p, start=n_ext)
            pltpu.sync_copy(vals_v, shv.at[pl.ds(sid * KPAD, KPAD)])
            pltpu.sync_copy(idx_v, shi.at[pl.ds(sid * KPAD, KPAD)])
            rem = jnp.max(_vmax_fold(t2_v, 0, G2))
            rb_v[pl.ds(0, L)] = jnp.full((L,), rem)
            pltpu.sync_copy(rb_v, shr.at[pl.ds(sid * L, L)])
            plsc.subcore_barrier()
            pltpu.sync_copy(shr, rall_v)
            gr = jnp.max(_vmax_fold(rall_v, 0, NW))
            acc = jnp.zeros((L,), jnp.int32)
            for c in range(KPAD // L):
                acc = acc + (vals_v[pl.ds(c * L, L)] > gr).astype(jnp.int32)
            cb_v[pl.ds(0, L)] = jnp.full((L,), jnp.sum(acc))
            pltpu.sync_copy(cb_v, shc.at[pl.ds(sid * L, L)])
            plsc.subcore_barrier()
            pltpu.sync_copy(shc, call_v)
            tot = jnp.max(_vadd_fold(call_v, 0, NW))
            n_new = n_ext + ROUND
            return n_new, jnp.logical_and(tot < K, n_new + ROUND <= KPAD)

        lax.while_loop(lambda st: st[1], round_body, (0, True))

        @pl.when(sid == 0)
        def _merge():
            pltpu.sync_copy(shv, mv_v)
            pltpu.sync_copy(shi, mi_v)

            for g in range(MG1):
                mt1_v[pl.ds(g * L, L)] = _vmax_fold(mv_v, g * L * L, L)
            mt2 = _vmax_fold(mt1_v, 0, MG1)
            mt2_v[pl.ds(0, L)] = mt2

            def idx_of(p):
                dv = mi_v[pl.ds((p // L) * L, L)]
                return jnp.min(jnp.where(iota == (p % L), dv, BIG))

            _extract_topk(mv_v, mt1_v, mt2_v, 1, MG1, fv_v, fi_v, K, idx_of)

            zeros = jnp.zeros((L,), jnp.int32)
            for j in range(4):
                for c in range(KPAD // L):
                    p = c * L + iota
                    gidx = fi_v[pl.ds(c * L, L)]
                    eidx = (gidx // NUM_CLASSES) * 4 + j
                    spat_v[pl.ds(j * KPAD + c * L, L)] = jnp.where(p < K, eidx, 0)
            descrs = [
                pltpu.async_copy(loc_hbm.at[spat_v.at[pl.ds(j * KPAD, KPAD)]],
                                 rows_v.at[j], sem)
                for j in range(4)
            ]
            for dsc in descrs:
                dsc.wait()

            for c in range(KPAD // L):
                p = c * L + iota
                valid = p < K
                gidx = fi_v[pl.ds(c * L, L)]
                for j in range(4):
                    colv = rows_v[j, pl.ds(c * L, L)]
                    plsc.store_scatter(det_v, [p, zeros + j], colv * 4.0, mask=valid)
                sc = fv_v[pl.ds(c * L, L)]
                plsc.store_scatter(det_v, [p, zeros + 4], sc, mask=valid)
                clsf = (gidx % NUM_CLASSES).astype(jnp.float32)
                plsc.store_scatter(det_v, [p, zeros + 5], clsf, mask=valid)

            pltpu.sync_copy(det_v, out_hbm)


@jax.jit
def kernel(cls_pred, loc_pred):
    cls_flat = jnp.reshape(cls_pred, (N,))
    loc_flat = jnp.reshape(loc_pred, (N // NUM_CLASSES * 4,))

    run = pl.kernel(
        _decode_kernel,
        out_type=jax.ShapeDtypeStruct((K, 6), jnp.float32),
        mesh=plsc.VectorSubcoreMesh(core_axis_name="c", subcore_axis_name="s",
                                    num_cores=1),
        compiler_params=pltpu.CompilerParams(needs_layout_passes=False),
        scratch_types=[
            pltpu.VMEM((CHUNK,), jnp.float32),
            pltpu.VMEM((ROWS,), jnp.float32),
            pltpu.VMEM((G1,), jnp.float32),
            pltpu.VMEM((KPAD,), jnp.float32),
            pltpu.VMEM((KPAD,), jnp.int32),
            pltpu.VMEM_SHARED((MERGE,), jnp.float32),
            pltpu.VMEM_SHARED((MERGE,), jnp.int32),
            pltpu.VMEM_SHARED((NW * L,), jnp.float32),
            pltpu.VMEM_SHARED((NW * L,), jnp.int32),
            pltpu.VMEM((L,), jnp.float32),
            pltpu.VMEM((NW * L,), jnp.float32),
            pltpu.VMEM((L,), jnp.int32),
            pltpu.VMEM((NW * L,), jnp.int32),
            pltpu.VMEM((MERGE,), jnp.float32),
            pltpu.VMEM((MERGE,), jnp.int32),
            pltpu.VMEM((MG1 * L,), jnp.float32),
            pltpu.VMEM((L,), jnp.float32),
            pltpu.VMEM((KPAD,), jnp.float32),
            pltpu.VMEM((KPAD,), jnp.int32),
            pltpu.VMEM((4 * KPAD,), jnp.int32),
            pltpu.VMEM((4, KPAD), jnp.float32),
            pltpu.VMEM((K, 6), jnp.float32),
            pltpu.SemaphoreType.DMA,
        ],
    )
    det = run(cls_flat, loc_flat)
    return jnp.reshape(det, (1, K, 6))
